# Initial kernel scaffold; baseline (speedup 1.0000x reference)
import jax
import jax.numpy as jnp
from jax.experimental import pallas as pl


def kernel(g, node_feat, edge_weight, hidden_state, Wc_z, bc_z, Wc_r, bc_r, Wc_h, bc_h, Wl_z, bl_z, Wl_r, bl_r, Wl_h, bl_h, W_out, b_out):
    raise NotImplementedError("write your pallas kernel here")


# speedup vs baseline: 24.4071x; 24.4071x over previous
import functools

import jax
import jax.numpy as jnp
from jax import lax
from jax.experimental import pallas as pl
from jax.experimental.pallas import tpu as pltpu
from jax.experimental.pallas import tpu_sc as plsc

NC = 2
NS = 16
NW = NC * NS
CH = 128
LANES = 16


def _deg_kernel(np_, epw, cpw, rpt):
    def body(dst_hbm, ew_hbm, deg_hbm, idx_v, val_v, zbuf, deg_sh):
        cid = lax.axis_index("c")
        sid = lax.axis_index("s")
        wid = cid * NS + sid
        zv = jnp.zeros((LANES,), jnp.float32)

        def zero_buf(j, c):
            zbuf[pl.ds(j * LANES, LANES)] = zv
            return c
        lax.fori_loop(0, rpt // LANES, zero_buf, 0)
        pltpu.sync_copy(zbuf, deg_sh.at[pl.ds(sid * rpt, rpt)])
        plsc.subcore_barrier()

        def chunk(j, c):
            base = wid * epw + j * CH
            pltpu.sync_copy(dst_hbm.at[pl.ds(base, CH)], idx_v)
            pltpu.sync_copy(ew_hbm.at[pl.ds(base, CH)], val_v)
            pltpu.sync_copy(val_v, deg_sh.at[idx_v], add=True)
            return c
        lax.fori_loop(0, cpw, chunk, 0)
        plsc.subcore_barrier()
        pltpu.sync_copy(deg_sh.at[pl.ds(sid * rpt, rpt)],
                        deg_hbm.at[cid, pl.ds(sid * rpt, rpt)])

    return pl.kernel(
        body,
        out_type=jax.ShapeDtypeStruct((NC, np_), jnp.float32),
        mesh=plsc.VectorSubcoreMesh(core_axis_name="c", subcore_axis_name="s"),
        scratch_types=[
            pltpu.VMEM((CH,), jnp.int32),
            pltpu.VMEM((CH,), jnp.float32),
            pltpu.VMEM((rpt,), jnp.float32),
            pltpu.VMEM_SHARED((np_,), jnp.float32),
        ],
    )


def _scatter_kernel(np_, feat, epw, cpw, rpt):
    def body(src_hbm, dst_hbm, ew_hbm, xs_hbm, s_hbm,
             sidx, didx, ewv, rows, zrow, s_sh, sem):
        cid = lax.axis_index("c")
        sid = lax.axis_index("s")
        wid = cid * NS + sid
        zv = jnp.zeros((LANES,), jnp.float32)
        for rr in range(LANES):
            for f in range(feat // LANES):
                zrow[rr, pl.ds(f * LANES, LANES)] = zv

        def zero_rows(j, c):
            pltpu.sync_copy(zrow, s_sh.at[pl.ds(sid * rpt + j * LANES, LANES)])
            return c
        lax.fori_loop(0, rpt // LANES, zero_rows, 0)
        plsc.subcore_barrier()

        def chunk(j, c):
            base = wid * epw + j * CH
            pltpu.sync_copy(src_hbm.at[pl.ds(base, CH)], sidx)
            pltpu.sync_copy(dst_hbm.at[pl.ds(base, CH)], didx)
            pltpu.sync_copy(ew_hbm.at[pl.ds(base, CH)], ewv)
            pltpu.async_copy(xs_hbm.at[sidx], rows, sem).wait()

            def scale(r16, cc):
                ew16 = ewv[pl.ds(r16 * LANES, LANES)]
                for k in range(LANES):
                    s = ew16[k]
                    row = r16 * LANES + k
                    for f in range(feat // LANES):
                        sl = pl.ds(f * LANES, LANES)
                        rows[row, sl] = rows[row, sl] * s
                return cc
            lax.fori_loop(0, CH // LANES, scale, 0)
            pltpu.sync_copy(rows, s_sh.at[didx], add=True)
            return c
        lax.fori_loop(0, cpw, chunk, 0)
        plsc.subcore_barrier()
        pltpu.sync_copy(s_sh.at[pl.ds(sid * rpt, rpt)],
                        s_hbm.at[cid, pl.ds(sid * rpt, rpt)])

    return pl.kernel(
        body,
        out_type=jax.ShapeDtypeStruct((NC, np_, feat), jnp.float32),
        mesh=plsc.VectorSubcoreMesh(core_axis_name="c", subcore_axis_name="s"),
        scratch_types=[
            pltpu.VMEM((CH,), jnp.int32),
            pltpu.VMEM((CH,), jnp.int32),
            pltpu.VMEM((CH,), jnp.float32),
            pltpu.VMEM((CH, feat), jnp.float32),
            pltpu.VMEM((LANES, feat), jnp.float32),
            pltpu.VMEM_SHARED((np_, feat), jnp.float32),
            pltpu.SemaphoreType.DMA,
        ],
    )


def _prep_body(deg_ref, x_ref, dinv_ref, xs_ref):
    d = 1.0 + deg_ref[0] + deg_ref[1]
    dinv = lax.rsqrt(d)
    dinv_ref[...] = dinv
    xs_ref[...] = x_ref[...] * dinv


def _dense_body(hid, s_ref, xs_ref, dinv_ref, h0_ref,
                wcz, wcr, wch, wlz, wlr, wlh, wout,
                bcz, bcr, bch, blz, blr, blh, bout,
                y_ref, h_ref, wz1, wr1, wh1, bz, br, bh):
    f32 = jnp.float32

    @pl.when(pl.program_id(0) == 0)
    def _():
        wz1[...] = jnp.dot(wcz[...], wlz[0:hid, :], preferred_element_type=f32)
        wr1[...] = jnp.dot(wcr[...], wlr[0:hid, :], preferred_element_type=f32)
        wh1[...] = jnp.dot(wch[...], wlh[0:hid, :], preferred_element_type=f32)
        bz[...] = jnp.dot(bcz[...], wlz[0:hid, :], preferred_element_type=f32) + blz[...]
        br[...] = jnp.dot(bcr[...], wlr[0:hid, :], preferred_element_type=f32) + blr[...]
        bh[...] = jnp.dot(bch[...], wlh[0:hid, :], preferred_element_type=f32) + blh[...]

    p = dinv_ref[...] * (s_ref[0] + s_ref[1] + xs_ref[...])
    h0 = h0_ref[...]
    zl = (jnp.dot(p, wz1[...], preferred_element_type=f32)
          + jnp.dot(h0, wlz[hid:2 * hid, :], preferred_element_type=f32) + bz[...])
    z = jax.nn.sigmoid(zl)
    rl = (jnp.dot(p, wr1[...], preferred_element_type=f32)
          + jnp.dot(h0, wlr[hid:2 * hid, :], preferred_element_type=f32) + br[...])
    r = jax.nn.sigmoid(rl)
    hl = (jnp.dot(p, wh1[...], preferred_element_type=f32)
          + jnp.dot(h0 * r, wlh[hid:2 * hid, :], preferred_element_type=f32) + bh[...])
    ht = jnp.tanh(hl)
    h = z * h0 + (1.0 - z) * ht
    h_ref[...] = h
    y_ref[...] = jnp.dot(jnp.maximum(h, 0.0), wout[...],
                         preferred_element_type=f32) + bout[...]


def kernel(g, node_feat, edge_weight, hidden_state,
           Wc_z, bc_z, Wc_r, bc_r, Wc_h, bc_h,
           Wl_z, bl_z, Wl_r, bl_r, Wl_h, bl_h, W_out, b_out):
    f32 = jnp.float32
    n, feat = node_feat.shape
    hid = hidden_state.shape[1]
    e = g.shape[1]

    npad = -(-n // (NS * LANES * 8)) * (NS * LANES * 8)
    epw = -(-e // (NW * CH)) * CH
    ep = epw * NW
    cpw = epw // CH
    rpt = npad // NS

    src = jnp.concatenate([g[0], jnp.zeros((ep - e,), g.dtype)])
    dst = jnp.concatenate([g[1], jnp.zeros((ep - e,), g.dtype)])
    ew = jnp.concatenate([edge_weight, jnp.zeros((ep - e,), f32)])
    x_pad = jnp.concatenate([node_feat, jnp.zeros((npad - n, feat), f32)])
    h0_pad = jnp.concatenate([hidden_state, jnp.zeros((npad - n, hid), f32)])

    deg_p = _deg_kernel(npad, epw, cpw, rpt)(dst, ew)
    deg_col = deg_p.reshape(NC, npad, 1)

    nb = 10
    blk = npad // nb
    dinv, xs = pl.pallas_call(
        _prep_body,
        grid=(nb,),
        in_specs=[
            pl.BlockSpec((NC, blk, 1), lambda i: (0, i, 0)),
            pl.BlockSpec((blk, feat), lambda i: (i, 0)),
        ],
        out_specs=[
            pl.BlockSpec((blk, 1), lambda i: (i, 0)),
            pl.BlockSpec((blk, feat), lambda i: (i, 0)),
        ],
        out_shape=[
            jax.ShapeDtypeStruct((npad, 1), f32),
            jax.ShapeDtypeStruct((npad, feat), f32),
        ],
    )(deg_col, x_pad)

    s_p = _scatter_kernel(npad, feat, epw, cpw, rpt)(src, dst, ew, xs)

    full = lambda shape: pl.BlockSpec(shape, lambda i: tuple(0 for _ in shape))
    y_pad, h_pad = pl.pallas_call(
        functools.partial(_dense_body, hid),
        grid=(nb,),
        in_specs=[
            pl.BlockSpec((NC, blk, feat), lambda i: (0, i, 0)),
            pl.BlockSpec((blk, feat), lambda i: (i, 0)),
            pl.BlockSpec((blk, 1), lambda i: (i, 0)),
            pl.BlockSpec((blk, hid), lambda i: (i, 0)),
            full((feat, hid)), full((feat, hid)), full((feat, hid)),
            full((2 * hid, hid)), full((2 * hid, hid)), full((2 * hid, hid)),
            full((hid, feat)),
            full((1, hid)), full((1, hid)), full((1, hid)),
            full((1, hid)), full((1, hid)), full((1, hid)),
            full((1, feat)),
        ],
        out_specs=[
            pl.BlockSpec((blk, feat), lambda i: (i, 0)),
            pl.BlockSpec((blk, hid), lambda i: (i, 0)),
        ],
        out_shape=[
            jax.ShapeDtypeStruct((npad, feat), f32),
            jax.ShapeDtypeStruct((npad, hid), f32),
        ],
        scratch_shapes=[
            pltpu.VMEM((feat, hid), f32), pltpu.VMEM((feat, hid), f32),
            pltpu.VMEM((feat, hid), f32),
            pltpu.VMEM((1, hid), f32), pltpu.VMEM((1, hid), f32),
            pltpu.VMEM((1, hid), f32),
        ],
    )(s_p, xs, dinv, h0_pad,
      Wc_z, Wc_r, Wc_h, Wl_z, Wl_r, Wl_h, W_out,
      bc_z.reshape(1, hid), bc_r.reshape(1, hid), bc_h.reshape(1, hid),
      bl_z.reshape(1, hid), bl_r.reshape(1, hid), bl_h.reshape(1, hid),
      b_out.reshape(1, feat))

    return y_pad[:n], h_pad[:n]
